# baseline (speedup 1.0000x reference)
import functools

import jax
import jax.numpy as jnp
from jax.experimental import pallas as pl
from jax.experimental.pallas import tpu as pltpu

_NB = 8


def _se_kernel(x_ref, w1t_ref, b1_ref, w2t_ref, b2_ref, o_ref, *, inv_hw):
    xv = x_ref[...]
    xf = xv.astype(jnp.float32)
    pooled = jnp.sum(xf, axis=-1) * inv_hw
    h = jnp.maximum(
        jnp.dot(pooled, w1t_ref[...],
                preferred_element_type=jnp.float32) + b1_ref[...], 0.0)
    s = jax.nn.sigmoid(
        jnp.dot(h, w2t_ref[...],
                preferred_element_type=jnp.float32) + b2_ref[...])
    o_ref[...] = (xf * s[:, :, None]).astype(jnp.bfloat16)


def kernel(x, w1, b1, w2, b2):
    N, C, H, W = x.shape
    Ch = w1.shape[0]
    HW = H * W
    xb = x.reshape(N, C, HW).astype(jnp.bfloat16)
    w1t = w1.T
    w2t = w2.T
    b1r = b1.reshape(1, Ch)
    b2r = b2.reshape(1, C)

    nb = _NB
    out_bf = pl.pallas_call(
        functools.partial(_se_kernel, inv_hw=1.0 / HW),
        out_shape=jax.ShapeDtypeStruct((N, C, HW), jnp.bfloat16),
        grid=(N // nb,),
        in_specs=[
            pl.BlockSpec((nb, C, HW), lambda n: (n, 0, 0)),
            pl.BlockSpec((C, Ch), lambda n: (0, 0)),
            pl.BlockSpec((1, Ch), lambda n: (0, 0)),
            pl.BlockSpec((Ch, C), lambda n: (0, 0)),
            pl.BlockSpec((1, C), lambda n: (0, 0)),
        ],
        out_specs=pl.BlockSpec((nb, C, HW), lambda n: (n, 0, 0)),
        compiler_params=pltpu.CompilerParams(
            dimension_semantics=("parallel",),
            vmem_limit_bytes=60 << 20),
        cost_estimate=pl.CostEstimate(
            flops=int(4 * N * C * Ch + 2 * N * C * HW),
            transcendentals=int(N * C),
            bytes_accessed=int(2 * N * C * HW * 2),
        ),
    )(xb, w1t, b1r, w2t, b2r)
    return out_bf.astype(jnp.float32).reshape(N, C, H, W)


# speedup vs baseline: 1.0051x; 1.0051x over previous
import functools

import jax
import jax.numpy as jnp
from jax.experimental import pallas as pl
from jax.experimental.pallas import tpu as pltpu

_NB = 8


def _se_kernel(x_ref, w1t_ref, b1_ref, w2t_ref, b2_ref, o_ref, *, inv_hw):
    xv = x_ref[...]
    pooled = jnp.sum(xv, axis=-1, dtype=jnp.float32) * inv_hw
    h = jnp.maximum(
        jnp.dot(pooled, w1t_ref[...],
                preferred_element_type=jnp.float32) + b1_ref[...], 0.0)
    s = jax.nn.sigmoid(
        jnp.dot(h, w2t_ref[...],
                preferred_element_type=jnp.float32) + b2_ref[...])
    o_ref[...] = xv * s.astype(jnp.bfloat16)[:, :, None]


def kernel(x, w1, b1, w2, b2):
    N, C, H, W = x.shape
    Ch = w1.shape[0]
    HW = H * W
    xb = x.reshape(N, C, HW).astype(jnp.bfloat16)
    w1t = w1.T
    w2t = w2.T
    b1r = b1.reshape(1, Ch)
    b2r = b2.reshape(1, C)

    nb = _NB
    out_bf = pl.pallas_call(
        functools.partial(_se_kernel, inv_hw=1.0 / HW),
        out_shape=jax.ShapeDtypeStruct((N, C, HW), jnp.bfloat16),
        grid=(N // nb,),
        in_specs=[
            pl.BlockSpec((nb, C, HW), lambda n: (n, 0, 0)),
            pl.BlockSpec((C, Ch), lambda n: (0, 0)),
            pl.BlockSpec((1, Ch), lambda n: (0, 0)),
            pl.BlockSpec((Ch, C), lambda n: (0, 0)),
            pl.BlockSpec((1, C), lambda n: (0, 0)),
        ],
        out_specs=pl.BlockSpec((nb, C, HW), lambda n: (n, 0, 0)),
        compiler_params=pltpu.CompilerParams(
            dimension_semantics=("parallel",),
            vmem_limit_bytes=60 << 20),
        cost_estimate=pl.CostEstimate(
            flops=int(4 * N * C * Ch + 2 * N * C * HW),
            transcendentals=int(N * C),
            bytes_accessed=int(2 * N * C * HW * 2),
        ),
    )(xb, w1t, b1r, w2t, b2r)
    return out_bf.astype(jnp.float32).reshape(N, C, H, W)


# speedup vs baseline: 1.0104x; 1.0053x over previous
import functools

import jax
import jax.numpy as jnp
from jax.experimental import pallas as pl
from jax.experimental.pallas import tpu as pltpu

_NB = 16


def _se_kernel(x_ref, w1t_ref, b1_ref, w2t_ref, b2_ref, o_ref, *, inv_hw):
    xv = x_ref[...]
    pooled = jnp.sum(xv, axis=-1, dtype=jnp.float32) * inv_hw
    h = jnp.maximum(
        jnp.dot(pooled, w1t_ref[...],
                preferred_element_type=jnp.float32) + b1_ref[...], 0.0)
    s = jax.nn.sigmoid(
        jnp.dot(h, w2t_ref[...],
                preferred_element_type=jnp.float32) + b2_ref[...])
    o_ref[...] = xv * s.astype(jnp.bfloat16)[:, :, None]


def kernel(x, w1, b1, w2, b2):
    N, C, H, W = x.shape
    Ch = w1.shape[0]
    HW = H * W
    xb = x.reshape(N, C, HW).astype(jnp.bfloat16)
    w1t = w1.T
    w2t = w2.T
    b1r = b1.reshape(1, Ch)
    b2r = b2.reshape(1, C)

    nb = _NB
    out_bf = pl.pallas_call(
        functools.partial(_se_kernel, inv_hw=1.0 / HW),
        out_shape=jax.ShapeDtypeStruct((N, C, HW), jnp.bfloat16),
        grid=(N // nb,),
        in_specs=[
            pl.BlockSpec((nb, C, HW), lambda n: (n, 0, 0)),
            pl.BlockSpec((C, Ch), lambda n: (0, 0)),
            pl.BlockSpec((1, Ch), lambda n: (0, 0)),
            pl.BlockSpec((Ch, C), lambda n: (0, 0)),
            pl.BlockSpec((1, C), lambda n: (0, 0)),
        ],
        out_specs=pl.BlockSpec((nb, C, HW), lambda n: (n, 0, 0)),
        compiler_params=pltpu.CompilerParams(
            dimension_semantics=("parallel",),
            vmem_limit_bytes=60 << 20),
        cost_estimate=pl.CostEstimate(
            flops=int(4 * N * C * Ch + 2 * N * C * HW),
            transcendentals=int(N * C),
            bytes_accessed=int(2 * N * C * HW * 2),
        ),
    )(xb, w1t, b1r, w2t, b2r)
    return out_bf.astype(jnp.float32).reshape(N, C, H, W)
